# baseline (speedup 1.0000x reference)
import jax
import jax.numpy as jnp
from jax import lax
from jax.experimental import pallas as pl


_B, _C, _H, _W = 8, 1024, 24, 24
_HW = _H * _W
_ROWS = _C * _HW // 128
_NG = 64
_GR = _ROWS // _NG
_BIG = 1 << 20


def _comb(va, ia, vb, ib):
    take_b = (vb > va) | ((vb == va) & (ib < ia))
    return jnp.where(take_b, vb, va), jnp.where(take_b, ib, ia)


def _body(x_ref, xo_ref, idx_ref):
    x = x_ref[0]
    xo_ref[0] = x
    x3 = x.reshape(_NG, _GR, 128)
    m = jnp.max(x3, axis=0)
    gi = lax.broadcasted_iota(jnp.int32, (_NG, _GR, 128), 0)
    g = jnp.min(jnp.where(x3 == m[None], gi, _NG), axis=0)

    fp = 128 * lax.broadcasted_iota(jnp.int32, (_GR, 128), 0) + \
        lax.broadcasted_iota(jnp.int32, (_GR, 128), 1)
    cslot = jnp.zeros((_GR, 128), jnp.int32)
    for t in range(1, 16):
        cslot += (fp >= t * _HW).astype(jnp.int32)
    idx = 16 * g + cslot

    v = m
    for half in (36, 18, 9):
        v, idx = _comb(v[:half], idx[:half], v[half:2 * half], idx[half:2 * half])

    av, ai = v[0:5], idx[0:5]
    pad_v = jnp.full((1, 64), -jnp.inf, jnp.float32)
    pad_i = jnp.full((1, 64), _BIG, jnp.int32)
    bv = jnp.concatenate(
        [v[4:9, 64:], jnp.concatenate([v[5:9, :64], pad_v], axis=0)], axis=1)
    bi = jnp.concatenate(
        [idx[4:9, 64:], jnp.concatenate([idx[5:9, :64], pad_i], axis=0)], axis=1)
    _, out_i = _comb(av, ai, bv, bi)
    idx_ref[0] = out_i


def kernel(inputs):
    x3 = inputs.reshape(_B, _ROWS, 128)
    x_out, idx = pl.pallas_call(
        _body,
        grid=(_B,),
        in_specs=[pl.BlockSpec((1, _ROWS, 128), lambda b: (b, 0, 0))],
        out_specs=[
            pl.BlockSpec((1, _ROWS, 128), lambda b: (b, 0, 0)),
            pl.BlockSpec((1, 5, 128), lambda b: (b, 0, 0)),
        ],
        out_shape=[
            jax.ShapeDtypeStruct((_B, _ROWS, 128), jnp.float32),
            jax.ShapeDtypeStruct((_B, 5, 128), jnp.int32),
        ],
    )(x3)
    idx = idx.reshape(_B, 5 * 128)[:, :_HW].reshape(_B, 1, _H, _W)
    return (x_out.reshape(_B, _C, _H, _W), idx)


# speedup vs baseline: 18.2200x; 18.2200x over previous
import jax
import jax.numpy as jnp
from jax import lax
from jax.experimental import pallas as pl


_B, _C, _H, _W = 8, 1024, 24, 24
_HW = _H * _W
_NT = _C // 128
_BIG = 1 << 20


def _body(x_ref, xo_ref, idx_ref):
    x = x_ref[0]
    xo_ref[0] = x
    m = x[:, 0:128]
    tidx = jnp.zeros((_HW, 128), jnp.int32)
    for t in range(1, _NT):
        xt = x[:, 128 * t:128 * (t + 1)]
        gt = xt > m
        m = jnp.where(gt, xt, m)
        tidx = jnp.where(gt, t, tidx)
    rowmax = jnp.max(m, axis=1, keepdims=True)
    lane = lax.broadcasted_iota(jnp.int32, (_HW, 128), 1)
    cand = jnp.where(m == rowmax, 128 * tidx + lane, _BIG)
    idx_ref[0, 0] = jnp.min(cand, axis=1)


def kernel(inputs):
    xt = inputs.transpose(0, 2, 3, 1).reshape(_B, _HW, _C)
    x_out, idx = pl.pallas_call(
        _body,
        grid=(_B,),
        in_specs=[pl.BlockSpec((1, _HW, _C), lambda b: (b, 0, 0))],
        out_specs=[
            pl.BlockSpec((1, _HW, _C), lambda b: (b, 0, 0)),
            pl.BlockSpec((1, 1, _HW), lambda b: (b, 0, 0)),
        ],
        out_shape=[
            jax.ShapeDtypeStruct((_B, _HW, _C), jnp.float32),
            jax.ShapeDtypeStruct((_B, 1, _HW), jnp.int32),
        ],
    )(xt)
    x_out = x_out.reshape(_B, _H, _W, _C).transpose(0, 3, 1, 2)
    return (x_out, idx.reshape(_B, 1, _H, _W))
